# baseline (speedup 1.0000x reference)
import functools
import math

import jax
import jax.numpy as jnp
from jax import lax
from jax.experimental import pallas as pl
from jax.experimental.pallas import tpu as pltpu
from jax.experimental.pallas import tpu_sc as plsc

D = 64
SCALE = math.sqrt(D)
CHUNK = 128


def _wid():
    return lax.axis_index("s") * plsc.get_sparse_core_info().num_cores + \
        lax.axis_index("c")


@functools.cache
def _make_k1(V):
    info = plsc.get_sparse_core_info()
    NW = info.num_cores * info.num_subcores
    nfull = V // CHUNK
    tail = V - nfull * CHUNK
    steps = nfull // NW + 2
    steps += steps % 2
    mesh = plsc.VectorSubcoreMesh(core_axis_name="c", subcore_axis_name="s")
    iota16 = lambda: lax.iota(jnp.int32, 16)

    @functools.partial(
        pl.kernel,
        mesh=mesh,
        compiler_params=pltpu.CompilerParams(
            use_tc_tiling_on_sc=True, needs_layout_passes=False),
        out_type=jax.ShapeDtypeStruct((V // 2, 2 * D), jnp.float32),
        scratch_types=[
            pltpu.VMEM((D, CHUNK), jnp.float32),
            pltpu.VMEM((D, CHUNK), jnp.float32),
            pltpu.VMEM((D, CHUNK), jnp.float32),
            pltpu.VMEM((D, CHUNK), jnp.float32),
            pltpu.VMEM((D, D), jnp.float32),
            pltpu.SemaphoreType.DMA,
            pltpu.SemaphoreType.DMA,
            pltpu.SemaphoreType.DMA,
            pltpu.SemaphoreType.DMA,
        ],
    )
    def k(tt_hbm, out_hbm, blk0, blk1, ob0, ob1, tail_blk,
          gsem0, gsem1, ssem0, ssem1):
        w = _wid()

        def col0(j):
            return (w + NW * j) * CHUNK

        def valid(j):
            return col0(j) < nfull * CHUNK

        def fire_in(j, blk, gsem):
            pltpu.async_copy(
                tt_hbm.at[:, pl.ds(pl.multiple_of(col0(j), CHUNK), CHUNK)],
                blk, gsem)

        def drain_in(j, blk, gsem):
            pltpu.make_async_copy(
                tt_hbm.at[:, pl.ds(pl.multiple_of(col0(j), CHUNK), CHUNK)],
                blk, gsem).wait()

        def fire_out(j, ob, ssem):
            pltpu.async_copy(
                ob, out_hbm.at[pl.ds(pl.multiple_of(col0(j) // 2, D), D)], ssem)

        def drain_out(ob, ssem):
            pltpu.make_async_copy(
                ob, out_hbm.at[pl.ds(0, D)], ssem).wait()

        def transpose(blk, ob):
            def body(g, _):
                for k16 in range(16):
                    p = g * 16 + k16
                    orow = g * 8 + (k16 // 2)
                    for t in range(D // 16):
                        vec = plsc.load_gather(
                            blk, [iota16() + 16 * t, jnp.full((16,), p, jnp.int32)])
                        ob[orow, pl.ds((k16 % 2) * D + 16 * t, 16)] = vec * SCALE
                return 0

            lax.fori_loop(0, CHUNK // 16, body, 0)

        def halfstep(j, first, blk, gsem, ob, ssem, oblk, ogsem, oob, ossem):
            @pl.when(valid(j + 1))
            def _():
                fire_in(j + 1, oblk, ogsem)

            @pl.when(valid(j))
            def _():
                drain_in(j, blk, gsem)
                transpose(blk, ob)

            if first:
                @pl.when(jnp.logical_and(j >= 1, valid(j - 1)))
                def _():
                    drain_out(oob, ossem)
            else:
                @pl.when(valid(j - 1))
                def _():
                    drain_out(oob, ossem)

            @pl.when(valid(j))
            def _():
                fire_out(j, ob, ssem)

        fire_in(0, blk0, gsem0)

        def body(t, _):
            halfstep(2 * t, True, blk0, gsem0, ob0, ssem0, blk1, gsem1, ob1, ssem1)
            halfstep(2 * t + 1, False, blk1, gsem1, ob1, ssem1, blk0, gsem0, ob0, ssem0)
            return 0

        lax.fori_loop(0, steps // 2, body, 0)
        @pl.when(valid(steps - 1))
        def _():
            drain_out(ob1 if (steps - 1) % 2 else ob0,
                      ssem1 if (steps - 1) % 2 else ssem0)

        if tail:
            @pl.when(w == 0)
            def _():
                pltpu.sync_copy(
                    tt_hbm.at[:, pl.ds(nfull * CHUNK, tail)], tail_blk)

                def tbody(g, _):
                    for k16 in range(16):
                        p = g * 16 + k16
                        orow = g * 8 + (k16 // 2)
                        for t in range(D // 16):
                            vec = plsc.load_gather(
                                tail_blk,
                                [iota16() + 16 * t, jnp.full((16,), p, jnp.int32)])
                            ob0[orow, pl.ds((k16 % 2) * D + 16 * t, 16)] = vec * SCALE
                    return 0

                lax.fori_loop(0, tail // 16, tbody, 0)
                pltpu.sync_copy(
                    ob0.at[pl.ds(0, tail // 2)],
                    out_hbm.at[pl.ds(nfull * CHUNK // 2, tail // 2)])

    return k


@functools.cache
def _make_k2(N, VP):
    info = plsc.get_sparse_core_info()
    NW = info.num_cores * info.num_subcores
    nchunks = N // CHUNK
    steps = nchunks // NW
    assert N % (CHUNK * NW) == 0 and steps % 2 == 0
    mesh = plsc.VectorSubcoreMesh(core_axis_name="c", subcore_axis_name="s")

    @functools.partial(
        pl.kernel,
        mesh=mesh,
        compiler_params=pltpu.CompilerParams(
            use_tc_tiling_on_sc=True, needs_layout_passes=False),
        out_type=jax.ShapeDtypeStruct((N // 2, 2 * D), jnp.float32),
        scratch_types=[
            pltpu.VMEM((steps, CHUNK), jnp.int32),
            pltpu.VMEM((steps, CHUNK), jnp.int32),
            pltpu.VMEM((CHUNK, 2 * D), jnp.float32),
            pltpu.VMEM((CHUNK, 2 * D), jnp.float32),
            pltpu.VMEM((D, CHUNK), jnp.float32),
            pltpu.VMEM((D, CHUNK), jnp.float32),
            pltpu.SemaphoreType.DMA,
            pltpu.SemaphoreType.DMA,
            pltpu.SemaphoreType.DMA,
            pltpu.SemaphoreType.DMA,
        ],
    )
    def k(xpair_hbm, xoff_hbm, tpair_hbm, out_hbm,
          pair_v, off_v, buf0, buf1, sel0, sel1,
          gsem0, gsem1, ssem0, ssem1):
        w = _wid()
        chunk_base = w * steps
        cb8 = pl.multiple_of(chunk_base, 8)
        pltpu.sync_copy(xpair_hbm.at[pl.ds(cb8, steps)], pair_v)
        pltpu.sync_copy(xoff_hbm.at[pl.ds(cb8, steps)], off_v)

        def fire_gather(j, buf, gsem):
            pltpu.async_copy(tpair_hbm.at[pair_v.at[j]], buf, gsem)

        def drain_gather(j, buf, gsem):
            pltpu.make_async_copy(tpair_hbm.at[pair_v.at[j]], buf, gsem).wait()

        def fire_store(j, sel, ssem):
            pltpu.async_copy(
                sel,
                out_hbm.at[pl.ds(pl.multiple_of((chunk_base + j) * D, D), D)],
                ssem)

        def drain_store(sel, ssem):
            pltpu.make_async_copy(sel, out_hbm.at[pl.ds(0, D)], ssem).wait()

        def select(j, buf, sel):
            def body(g, _):
                offs = off_v[j, pl.ds(g * 16, 16)]
                for k16 in range(16):
                    off = offs[k16]
                    r = g * 16 + k16
                    orow = g * 8 + (k16 // 2)
                    for t in range(D // 16):
                        sel[orow, pl.ds((k16 % 2) * D + 16 * t, 16)] = \
                            buf[r, pl.ds(off + 16 * t, 16)]
                return 0

            lax.fori_loop(0, CHUNK // 16, body, 0)

        def halfstep(j, first, buf, gsem, sel, ssem, obuf, ogsem, osel, ossem):
            drain_gather(j, buf, gsem)
            if first:
                @pl.when(j >= 1)
                def _():
                    drain_store(osel, ossem)
            else:
                drain_store(osel, ossem)

            @pl.when(j + 1 < steps)
            def _():
                fire_gather(j + 1, obuf, ogsem)

            select(j, buf, sel)
            fire_store(j, sel, ssem)

        fire_gather(0, buf0, gsem0)

        def body(t, _):
            halfstep(2 * t, True, buf0, gsem0, sel0, ssem0, buf1, gsem1, sel1, ssem1)
            halfstep(2 * t + 1, False, buf1, gsem1, sel1, ssem1, buf0, gsem0, sel0, ssem0)
            return 0

        lax.fori_loop(0, steps // 2, body, 0)
        drain_store(sel1, ssem1)

    return k


def kernel(x, table):
    B, L = x.shape
    V = table.shape[0]
    N = B * L
    tpair = _make_k1(V)(jnp.swapaxes(table, 0, 1))
    xi = x.astype(jnp.int32)
    xpair = (xi >> 1).reshape(N // CHUNK, CHUNK)
    xoff = ((xi & 1) * D).reshape(N // CHUNK, CHUNK)
    out = _make_k2(N, V // 2)(xpair, xoff, tpair)
    return out.reshape(B, L, D)


# speedup vs baseline: 1.3282x; 1.3282x over previous
import functools
import math

import jax
import jax.numpy as jnp
from jax import lax
from jax.experimental import pallas as pl
from jax.experimental.pallas import tpu as pltpu
from jax.experimental.pallas import tpu_sc as plsc

D = 64
SCALE = math.sqrt(D)
CHUNK = 128
K1_STRIDE = 144
K2_STRIDE = 80


def _wid():
    return lax.axis_index("s") * plsc.get_sparse_core_info().num_cores + \
        lax.axis_index("c")


def _iota16():
    return lax.iota(jnp.int32, 16)


@functools.cache
def _make_k1(V):
    info = plsc.get_sparse_core_info()
    NW = info.num_cores * info.num_subcores
    nfull = V // CHUNK
    tail = V - nfull * CHUNK
    steps = nfull // NW + 2
    steps += steps % 2
    mesh = plsc.VectorSubcoreMesh(core_axis_name="c", subcore_axis_name="s")

    @functools.partial(
        pl.kernel,
        mesh=mesh,
        compiler_params=pltpu.CompilerParams(
            use_tc_tiling_on_sc=True, needs_layout_passes=False),
        out_type=jax.ShapeDtypeStruct((V // 2, 2 * D), jnp.float32),
        scratch_types=[
            pltpu.VMEM((D, CHUNK), jnp.float32),
            pltpu.VMEM((D, CHUNK), jnp.float32),
            pltpu.VMEM((D, CHUNK), jnp.float32),
            pltpu.VMEM((D, CHUNK), jnp.float32),
            pltpu.VMEM((D, D), jnp.float32),
            pltpu.VMEM((D * K1_STRIDE,), jnp.float32),
            pltpu.SemaphoreType.DMA,
            pltpu.SemaphoreType.DMA,
            pltpu.SemaphoreType.DMA,
            pltpu.SemaphoreType.DMA,
        ],
    )
    def k(tt_hbm, out_hbm, blk0, blk1, ob0, ob1, tail_blk, skew,
          gsem0, gsem1, ssem0, ssem1):
        w = _wid()

        def col0(j):
            return (w + NW * j) * CHUNK

        def valid(j):
            return col0(j) < nfull * CHUNK

        def fire_in(j, blk, gsem):
            pltpu.async_copy(
                tt_hbm.at[:, pl.ds(pl.multiple_of(col0(j), CHUNK), CHUNK)],
                blk, gsem)

        def drain_in(j, blk, gsem):
            pltpu.make_async_copy(
                tt_hbm.at[:, pl.ds(pl.multiple_of(col0(j), CHUNK), CHUNK)],
                blk, gsem).wait()

        def fire_out(j, ob, ssem):
            pltpu.async_copy(
                ob, out_hbm.at[pl.ds(pl.multiple_of(col0(j) // 2, D), D)], ssem)

        def drain_out(ob, ssem):
            pltpu.make_async_copy(
                ob, out_hbm.at[pl.ds(0, D)], ssem).wait()

        def transpose(blk, ob, ncols):
            def skew_row(d, _):
                base = d * K1_STRIDE
                for t in range(ncols // 16):
                    v = blk[d, pl.ds(16 * t, 16)]
                    pos = base + ((16 * t + d) & 127)
                    plsc.store_scatter(skew, [_iota16() + pos], v)
                return 0

            lax.fori_loop(0, D, skew_row, 0)

            def degroup(g, _):
                for t in range(D // 16):
                    dv = _iota16() + 16 * t
                    for k16 in range(16):
                        hv = dv * K1_STRIDE + ((g * 16 + dv) & 127) + k16
                        vec = plsc.load_gather(skew, [hv])
                        ob[g * 8 + k16 // 2,
                           pl.ds((k16 % 2) * D + 16 * t, 16)] = vec * SCALE
                return 0

            lax.fori_loop(0, ncols // 16, degroup, 0)

        def halfstep(j, first, blk, gsem, ob, ssem, oblk, ogsem, oob, ossem):
            @pl.when(valid(j + 1))
            def _():
                fire_in(j + 1, oblk, ogsem)

            @pl.when(valid(j))
            def _():
                drain_in(j, blk, gsem)
                transpose(blk, ob, CHUNK)

            if first:
                @pl.when(jnp.logical_and(j >= 1, valid(j - 1)))
                def _():
                    drain_out(oob, ossem)
            else:
                @pl.when(valid(j - 1))
                def _():
                    drain_out(oob, ossem)

            @pl.when(valid(j))
            def _():
                fire_out(j, ob, ssem)

        fire_in(0, blk0, gsem0)

        def body(t, _):
            halfstep(2 * t, True, blk0, gsem0, ob0, ssem0, blk1, gsem1, ob1, ssem1)
            halfstep(2 * t + 1, False, blk1, gsem1, ob1, ssem1, blk0, gsem0, ob0, ssem0)
            return 0

        lax.fori_loop(0, steps // 2, body, 0)
        @pl.when(valid(steps - 1))
        def _():
            drain_out(ob1 if (steps - 1) % 2 else ob0,
                      ssem1 if (steps - 1) % 2 else ssem0)

        if tail:
            @pl.when(w == 0)
            def _():
                pltpu.sync_copy(
                    tt_hbm.at[:, pl.ds(nfull * CHUNK, tail)], tail_blk)
                transpose(tail_blk, ob0, tail)
                pltpu.sync_copy(
                    ob0.at[pl.ds(0, tail // 2)],
                    out_hbm.at[pl.ds(nfull * CHUNK // 2, tail // 2)])

    return k


@functools.cache
def _make_k2(B, L):
    info = plsc.get_sparse_core_info()
    NW = info.num_cores * info.num_subcores
    assert B // NW == CHUNK and L % 2 == 0
    steps = L
    mesh = plsc.VectorSubcoreMesh(core_axis_name="c", subcore_axis_name="s")

    @functools.partial(
        pl.kernel,
        mesh=mesh,
        compiler_params=pltpu.CompilerParams(
            use_tc_tiling_on_sc=True, needs_layout_passes=False),
        out_type=jax.ShapeDtypeStruct((L, D, B), jnp.float32),
        scratch_types=[
            pltpu.VMEM((L, CHUNK), jnp.int32),
            pltpu.VMEM((L, CHUNK), jnp.int32),
            pltpu.VMEM((CHUNK, 2 * D), jnp.float32),
            pltpu.VMEM((CHUNK, 2 * D), jnp.float32),
            pltpu.VMEM((D, CHUNK), jnp.float32),
            pltpu.VMEM((D, CHUNK), jnp.float32),
            pltpu.VMEM((CHUNK * K2_STRIDE,), jnp.float32),
            pltpu.SemaphoreType.DMA,
            pltpu.SemaphoreType.DMA,
            pltpu.SemaphoreType.DMA,
            pltpu.SemaphoreType.DMA,
        ],
    )
    def k(xpair_hbm, xoff_hbm, tpair_hbm, out_hbm,
          pair_v, off_v, buf0, buf1, tb0, tb1, skew,
          gsem0, gsem1, ssem0, ssem1):
        w = _wid()
        bcol = pl.multiple_of(w * CHUNK, CHUNK)
        pltpu.sync_copy(xpair_hbm.at[:, pl.ds(bcol, CHUNK)], pair_v)
        pltpu.sync_copy(xoff_hbm.at[:, pl.ds(bcol, CHUNK)], off_v)

        def fire_gather(l, buf, gsem):
            pltpu.async_copy(tpair_hbm.at[pair_v.at[l]], buf, gsem)

        def drain_gather(l, buf, gsem):
            pltpu.make_async_copy(tpair_hbm.at[pair_v.at[l]], buf, gsem).wait()

        def fire_store(l, tb, ssem):
            pltpu.async_copy(tb, out_hbm.at[l, :, pl.ds(bcol, CHUNK)], ssem)

        def drain_store(tb, ssem):
            pltpu.make_async_copy(
                tb, out_hbm.at[0, :, pl.ds(bcol, CHUNK)], ssem).wait()

        def sel_transpose(l, buf, tb):
            def skew_row(g, _):
                offs = off_v[l, pl.ds(g * 16, 16)]
                for k16 in range(16):
                    off = offs[k16]
                    b = g * 16 + k16
                    base = b * K2_STRIDE
                    for t in range(D // 16):
                        v = buf[b, pl.ds(off + 16 * t, 16)]
                        pos = base + ((16 * t + b) & 63)
                        plsc.store_scatter(skew, [_iota16() + pos], v)
                return 0

            lax.fori_loop(0, CHUNK // 16, skew_row, 0)

            def degroup(u, _):
                bv = _iota16() + 16 * u
                for d16 in range(0, D, 16):
                    hv = bv * K2_STRIDE + ((d16 + bv) & 63)
                    for dm in range(16):
                        vec = plsc.load_gather(skew, [hv + dm])
                        tb[d16 + dm, pl.ds(16 * u, 16)] = vec
                return 0

            lax.fori_loop(0, CHUNK // 16, degroup, 0)

        def halfstep(l, first, buf, gsem, tb, ssem, obuf, ogsem, otb, ossem):
            drain_gather(l, buf, gsem)
            if first:
                @pl.when(l >= 1)
                def _():
                    drain_store(otb, ossem)
            else:
                drain_store(otb, ossem)

            @pl.when(l + 1 < steps)
            def _():
                fire_gather(l + 1, obuf, ogsem)

            sel_transpose(l, buf, tb)
            fire_store(l, tb, ssem)

        fire_gather(0, buf0, gsem0)

        def body(t, _):
            halfstep(2 * t, True, buf0, gsem0, tb0, ssem0, buf1, gsem1, tb1, ssem1)
            halfstep(2 * t + 1, False, buf1, gsem1, tb1, ssem1, buf0, gsem0, tb0, ssem0)
            return 0

        lax.fori_loop(0, steps // 2, body, 0)
        drain_store(tb1, ssem1)

    return k


def kernel(x, table):
    B, L = x.shape
    V = table.shape[0]
    tpair = _make_k1(V)(jnp.swapaxes(table, 0, 1))
    xt = jnp.swapaxes(x, 0, 1).astype(jnp.int32)
    xpair = xt >> 1
    xoff = (xt & 1) * D
    out = _make_k2(B, L)(xpair, xoff, tpair)
    return jnp.transpose(out, (2, 0, 1))


# speedup vs baseline: 2.9185x; 2.1973x over previous
import functools
import math

import jax
import jax.numpy as jnp
from jax import lax
from jax.experimental import pallas as pl
from jax.experimental.pallas import tpu as pltpu
from jax.experimental.pallas import tpu_sc as plsc

D = 64
SCALE = math.sqrt(D)
CHUNK = 128
K1_STRIDE = 144
K2_STRIDE = 80


def _wid():
    return lax.axis_index("s") * plsc.get_sparse_core_info().num_cores + \
        lax.axis_index("c")


def _iota16():
    return lax.iota(jnp.int32, 16)


@functools.cache
def _make_k1(V):
    info = plsc.get_sparse_core_info()
    NW = info.num_cores * info.num_subcores
    nfull = V // CHUNK
    tail = V - nfull * CHUNK
    steps = nfull // NW + 2
    steps += steps % 2
    mesh = plsc.VectorSubcoreMesh(core_axis_name="c", subcore_axis_name="s")

    @functools.partial(
        pl.kernel,
        mesh=mesh,
        compiler_params=pltpu.CompilerParams(
            use_tc_tiling_on_sc=True, needs_layout_passes=False),
        out_type=jax.ShapeDtypeStruct((V // 2, 2 * D), jnp.float32),
        scratch_types=[
            pltpu.VMEM((D, CHUNK), jnp.float32),
            pltpu.VMEM((D, CHUNK), jnp.float32),
            pltpu.VMEM((D, CHUNK), jnp.float32),
            pltpu.VMEM((D, CHUNK), jnp.float32),
            pltpu.VMEM((D, D), jnp.float32),
            pltpu.VMEM((D * K1_STRIDE,), jnp.float32),
            pltpu.SemaphoreType.DMA,
            pltpu.SemaphoreType.DMA,
            pltpu.SemaphoreType.DMA,
            pltpu.SemaphoreType.DMA,
        ],
    )
    def k(tt_hbm, out_hbm, blk0, blk1, ob0, ob1, tail_blk, skew,
          gsem0, gsem1, ssem0, ssem1):
        w = _wid()

        def col0(j):
            return (w + NW * j) * CHUNK

        def valid(j):
            return col0(j) < nfull * CHUNK

        def fire_in(j, blk, gsem):
            pltpu.async_copy(
                tt_hbm.at[:, pl.ds(pl.multiple_of(col0(j), CHUNK), CHUNK)],
                blk, gsem)

        def drain_in(j, blk, gsem):
            pltpu.make_async_copy(
                tt_hbm.at[:, pl.ds(pl.multiple_of(col0(j), CHUNK), CHUNK)],
                blk, gsem).wait()

        def fire_out(j, ob, ssem):
            pltpu.async_copy(
                ob, out_hbm.at[pl.ds(pl.multiple_of(col0(j) // 2, D), D)], ssem)

        def drain_out(ob, ssem):
            pltpu.make_async_copy(
                ob, out_hbm.at[pl.ds(0, D)], ssem).wait()

        def transpose(blk, ob, ncols):
            def skew_row(d, _):
                base = d * K1_STRIDE
                nt = ncols // 16
                for t0 in range(0, nt, 4):
                    vs = [blk[d, pl.ds(16 * t, 16)]
                          for t in range(t0, min(t0 + 4, nt))]
                    for i, t in enumerate(range(t0, min(t0 + 4, nt))):
                        pos = base + ((16 * t + d) & 127)
                        plsc.store_scatter(skew, [_iota16() + pos], vs[i])
                return 0

            lax.fori_loop(0, D, skew_row, 0)

            def degroup(g, _):
                for t in range(D // 16):
                    dv = _iota16() + 16 * t
                    hbase = dv * K1_STRIDE + ((g * 16 + dv) & 127)
                    for k0 in range(0, 16, 4):
                        vecs = [plsc.load_gather(skew, [hbase + k16])
                                for k16 in range(k0, k0 + 4)]
                        for i, k16 in enumerate(range(k0, k0 + 4)):
                            ob[g * 8 + k16 // 2,
                               pl.ds((k16 % 2) * D + 16 * t, 16)] = vecs[i] * SCALE
                return 0

            lax.fori_loop(0, ncols // 16, degroup, 0)

        def halfstep(j, first, blk, gsem, ob, ssem, oblk, ogsem, oob, ossem):
            @pl.when(valid(j + 1))
            def _():
                fire_in(j + 1, oblk, ogsem)

            @pl.when(valid(j))
            def _():
                drain_in(j, blk, gsem)
                transpose(blk, ob, CHUNK)

            if first:
                @pl.when(jnp.logical_and(j >= 1, valid(j - 1)))
                def _():
                    drain_out(oob, ossem)
            else:
                @pl.when(valid(j - 1))
                def _():
                    drain_out(oob, ossem)

            @pl.when(valid(j))
            def _():
                fire_out(j, ob, ssem)

        fire_in(0, blk0, gsem0)

        def body(t, _):
            halfstep(2 * t, True, blk0, gsem0, ob0, ssem0, blk1, gsem1, ob1, ssem1)
            halfstep(2 * t + 1, False, blk1, gsem1, ob1, ssem1, blk0, gsem0, ob0, ssem0)
            return 0

        lax.fori_loop(0, steps // 2, body, 0)
        @pl.when(valid(steps - 1))
        def _():
            drain_out(ob1 if (steps - 1) % 2 else ob0,
                      ssem1 if (steps - 1) % 2 else ssem0)

        if tail:
            @pl.when(w == 0)
            def _():
                pltpu.sync_copy(
                    tt_hbm.at[:, pl.ds(nfull * CHUNK, tail)], tail_blk)
                transpose(tail_blk, ob0, tail)
                pltpu.sync_copy(
                    ob0.at[pl.ds(0, tail // 2)],
                    out_hbm.at[pl.ds(nfull * CHUNK // 2, tail // 2)])

    return k


@functools.cache
def _make_k2(B, L):
    info = plsc.get_sparse_core_info()
    NW = info.num_cores * info.num_subcores
    assert B // NW == CHUNK and L % 2 == 0
    steps = L
    mesh = plsc.VectorSubcoreMesh(core_axis_name="c", subcore_axis_name="s")

    @functools.partial(
        pl.kernel,
        mesh=mesh,
        compiler_params=pltpu.CompilerParams(
            use_tc_tiling_on_sc=True, needs_layout_passes=False),
        out_type=jax.ShapeDtypeStruct((L, D, B), jnp.float32),
        scratch_types=[
            pltpu.VMEM((L, CHUNK), jnp.int32),
            pltpu.VMEM((L, CHUNK), jnp.int32),
            pltpu.VMEM((CHUNK, 2 * D), jnp.float32),
            pltpu.VMEM((CHUNK, 2 * D), jnp.float32),
            pltpu.VMEM((D, CHUNK), jnp.float32),
            pltpu.VMEM((D, CHUNK), jnp.float32),
            pltpu.VMEM((CHUNK * K2_STRIDE,), jnp.float32),
            pltpu.SemaphoreType.DMA,
            pltpu.SemaphoreType.DMA,
            pltpu.SemaphoreType.DMA,
            pltpu.SemaphoreType.DMA,
        ],
    )
    def k(xpair_hbm, xoff_hbm, tpair_hbm, out_hbm,
          pair_v, off_v, buf0, buf1, tb0, tb1, skew,
          gsem0, gsem1, ssem0, ssem1):
        w = _wid()
        bcol = pl.multiple_of(w * CHUNK, CHUNK)
        pltpu.sync_copy(xpair_hbm.at[:, pl.ds(bcol, CHUNK)], pair_v)
        pltpu.sync_copy(xoff_hbm.at[:, pl.ds(bcol, CHUNK)], off_v)

        def fire_gather(l, buf, gsem):
            pltpu.async_copy(tpair_hbm.at[pair_v.at[l]], buf, gsem)

        def drain_gather(l, buf, gsem):
            pltpu.make_async_copy(tpair_hbm.at[pair_v.at[l]], buf, gsem).wait()

        def fire_store(l, tb, ssem):
            pltpu.async_copy(tb, out_hbm.at[l, :, pl.ds(bcol, CHUNK)], ssem)

        def drain_store(tb, ssem):
            pltpu.make_async_copy(
                tb, out_hbm.at[0, :, pl.ds(bcol, CHUNK)], ssem).wait()

        def sel_transpose(l, buf, tb):
            def skew_row(g, _):
                offs = off_v[l, pl.ds(g * 16, 16)]
                for k16 in range(16):
                    off = offs[k16]
                    b = g * 16 + k16
                    base = b * K2_STRIDE
                    vs = [buf[b, pl.ds(off + 16 * t, 16)]
                          for t in range(D // 16)]
                    for t in range(D // 16):
                        pos = base + ((16 * t + b) & 63)
                        plsc.store_scatter(skew, [_iota16() + pos], vs[t])
                return 0

            lax.fori_loop(0, CHUNK // 16, skew_row, 0)

            def degroup(u, _):
                bv = _iota16() + 16 * u
                for d16 in range(0, D, 16):
                    hv = bv * K2_STRIDE + ((d16 + bv) & 63)
                    for d0 in range(0, 16, 4):
                        vecs = [plsc.load_gather(skew, [hv + dm])
                                for dm in range(d0, d0 + 4)]
                        for i, dm in enumerate(range(d0, d0 + 4)):
                            tb[d16 + dm, pl.ds(16 * u, 16)] = vecs[i]
                return 0

            lax.fori_loop(0, CHUNK // 16, degroup, 0)

        def halfstep(l, first, buf, gsem, tb, ssem, obuf, ogsem, otb, ossem):
            drain_gather(l, buf, gsem)
            if first:
                @pl.when(l >= 1)
                def _():
                    drain_store(otb, ossem)
            else:
                drain_store(otb, ossem)

            @pl.when(l + 1 < steps)
            def _():
                fire_gather(l + 1, obuf, ogsem)

            sel_transpose(l, buf, tb)
            fire_store(l, tb, ssem)

        fire_gather(0, buf0, gsem0)

        def body(t, _):
            halfstep(2 * t, True, buf0, gsem0, tb0, ssem0, buf1, gsem1, tb1, ssem1)
            halfstep(2 * t + 1, False, buf1, gsem1, tb1, ssem1, buf0, gsem0, tb0, ssem0)
            return 0

        lax.fori_loop(0, steps // 2, body, 0)
        drain_store(tb1, ssem1)

    return k


def kernel(x, table):
    B, L = x.shape
    V = table.shape[0]
    tpair = _make_k1(V)(jnp.swapaxes(table, 0, 1))
    xt = jnp.swapaxes(x, 0, 1).astype(jnp.int32)
    xpair = xt >> 1
    xoff = (xt & 1) * D
    out = _make_k2(B, L)(xpair, xoff, tpair)
    return jnp.transpose(out, (2, 0, 1))


# speedup vs baseline: 3.3623x; 1.1521x over previous
import functools
import math

import jax
import jax.numpy as jnp
from jax import lax
from jax.experimental import pallas as pl
from jax.experimental.pallas import tpu as pltpu
from jax.experimental.pallas import tpu_sc as plsc

D = 64
SCALE = math.sqrt(D)
CHUNK = 128
K1_STRIDE = 144
K2_STRIDE = 80


def _wid():
    return lax.axis_index("s") * plsc.get_sparse_core_info().num_cores + \
        lax.axis_index("c")


def _iota16():
    return lax.iota(jnp.int32, 16)


@functools.cache
def _make_k1(V):
    info = plsc.get_sparse_core_info()
    NW = info.num_cores * info.num_subcores
    nfull = V // CHUNK
    tail = V - nfull * CHUNK
    steps = nfull // NW + 2
    steps += steps % 2
    mesh = plsc.VectorSubcoreMesh(core_axis_name="c", subcore_axis_name="s")

    @functools.partial(
        pl.kernel,
        mesh=mesh,
        compiler_params=pltpu.CompilerParams(
            use_tc_tiling_on_sc=True, needs_layout_passes=False),
        out_type=jax.ShapeDtypeStruct((V // 2, 2 * D), jnp.float32),
        scratch_types=[
            pltpu.VMEM((D, CHUNK), jnp.float32),
            pltpu.VMEM((D, CHUNK), jnp.float32),
            pltpu.VMEM((D, CHUNK), jnp.float32),
            pltpu.VMEM((D, CHUNK), jnp.float32),
            pltpu.VMEM((D, D), jnp.float32),
            pltpu.VMEM((D * K1_STRIDE,), jnp.float32),
            pltpu.SemaphoreType.DMA,
            pltpu.SemaphoreType.DMA,
            pltpu.SemaphoreType.DMA,
            pltpu.SemaphoreType.DMA,
        ],
    )
    def k(tt_hbm, out_hbm, blk0, blk1, ob0, ob1, tail_blk, skew,
          gsem0, gsem1, ssem0, ssem1):
        w = _wid()

        def col0(j):
            return (w + NW * j) * CHUNK

        def valid(j):
            return col0(j) < nfull * CHUNK

        def fire_in(j, blk, gsem):
            pltpu.async_copy(
                tt_hbm.at[:, pl.ds(pl.multiple_of(col0(j), CHUNK), CHUNK)],
                blk, gsem)

        def drain_in(j, blk, gsem):
            pltpu.make_async_copy(
                tt_hbm.at[:, pl.ds(pl.multiple_of(col0(j), CHUNK), CHUNK)],
                blk, gsem).wait()

        def fire_out(j, ob, ssem):
            pltpu.async_copy(
                ob, out_hbm.at[pl.ds(pl.multiple_of(col0(j) // 2, D), D)], ssem)

        def drain_out(ob, ssem):
            pltpu.make_async_copy(
                ob, out_hbm.at[pl.ds(0, D)], ssem).wait()

        def transpose(blk, ob, ncols):
            def skew_row(d, _):
                base = d * K1_STRIDE
                nt = ncols // 16
                for t0 in range(0, nt, 8):
                    vs = [blk[d, pl.ds(16 * t, 16)]
                          for t in range(t0, min(t0 + 8, nt))]
                    for i, t in enumerate(range(t0, min(t0 + 8, nt))):
                        pos = base + ((16 * t + d) & 127)
                        plsc.store_scatter(skew, [_iota16() + pos], vs[i])
                return 0

            lax.fori_loop(0, D, skew_row, 0)

            def degroup(g, _):
                for t in range(D // 16):
                    dv = _iota16() + 16 * t
                    hbase = dv * K1_STRIDE + ((g * 16 + dv) & 127)
                    for k0 in range(0, 16, 8):
                        vecs = [plsc.load_gather(skew, [hbase + k16])
                                for k16 in range(k0, k0 + 8)]
                        for i, k16 in enumerate(range(k0, k0 + 8)):
                            ob[g * 8 + k16 // 2,
                               pl.ds((k16 % 2) * D + 16 * t, 16)] = vecs[i] * SCALE
                return 0

            lax.fori_loop(0, ncols // 16, degroup, 0)

        def halfstep(j, first, blk, gsem, ob, ssem, oblk, ogsem, oob, ossem):
            @pl.when(valid(j + 1))
            def _():
                fire_in(j + 1, oblk, ogsem)

            @pl.when(valid(j))
            def _():
                drain_in(j, blk, gsem)
                transpose(blk, ob, CHUNK)

            if first:
                @pl.when(jnp.logical_and(j >= 1, valid(j - 1)))
                def _():
                    drain_out(oob, ossem)
            else:
                @pl.when(valid(j - 1))
                def _():
                    drain_out(oob, ossem)

            @pl.when(valid(j))
            def _():
                fire_out(j, ob, ssem)

        fire_in(0, blk0, gsem0)

        def body(t, _):
            halfstep(2 * t, True, blk0, gsem0, ob0, ssem0, blk1, gsem1, ob1, ssem1)
            halfstep(2 * t + 1, False, blk1, gsem1, ob1, ssem1, blk0, gsem0, ob0, ssem0)
            return 0

        lax.fori_loop(0, steps // 2, body, 0)
        @pl.when(valid(steps - 1))
        def _():
            drain_out(ob1 if (steps - 1) % 2 else ob0,
                      ssem1 if (steps - 1) % 2 else ssem0)

        if tail:
            @pl.when(w == 0)
            def _():
                pltpu.sync_copy(
                    tt_hbm.at[:, pl.ds(nfull * CHUNK, tail)], tail_blk)
                transpose(tail_blk, ob0, tail)
                pltpu.sync_copy(
                    ob0.at[pl.ds(0, tail // 2)],
                    out_hbm.at[pl.ds(nfull * CHUNK // 2, tail // 2)])

    return k


@functools.cache
def _make_k2(B, L):
    info = plsc.get_sparse_core_info()
    NW = info.num_cores * info.num_subcores
    assert B // NW == CHUNK and L % 2 == 0
    steps = L
    mesh = plsc.VectorSubcoreMesh(core_axis_name="c", subcore_axis_name="s")

    @functools.partial(
        pl.kernel,
        mesh=mesh,
        compiler_params=pltpu.CompilerParams(
            use_tc_tiling_on_sc=True, needs_layout_passes=False),
        out_type=jax.ShapeDtypeStruct((L, D, B), jnp.float32),
        scratch_types=[
            pltpu.VMEM((L, CHUNK), jnp.int32),
            pltpu.VMEM((L, CHUNK), jnp.int32),
            pltpu.VMEM((CHUNK, 2 * D), jnp.float32),
            pltpu.VMEM((CHUNK, 2 * D), jnp.float32),
            pltpu.VMEM((D, CHUNK), jnp.float32),
            pltpu.VMEM((D, CHUNK), jnp.float32),
            pltpu.VMEM((CHUNK * K2_STRIDE,), jnp.float32),
            pltpu.SemaphoreType.DMA,
            pltpu.SemaphoreType.DMA,
            pltpu.SemaphoreType.DMA,
            pltpu.SemaphoreType.DMA,
        ],
    )
    def k(xpair_hbm, xoff_hbm, tpair_hbm, out_hbm,
          pair_v, off_v, buf0, buf1, tb0, tb1, skew,
          gsem0, gsem1, ssem0, ssem1):
        w = _wid()
        bcol = pl.multiple_of(w * CHUNK, CHUNK)
        pltpu.sync_copy(xpair_hbm.at[:, pl.ds(bcol, CHUNK)], pair_v)
        pltpu.sync_copy(xoff_hbm.at[:, pl.ds(bcol, CHUNK)], off_v)

        def fire_gather(l, buf, gsem):
            pltpu.async_copy(tpair_hbm.at[pair_v.at[l]], buf, gsem)

        def drain_gather(l, buf, gsem):
            pltpu.make_async_copy(tpair_hbm.at[pair_v.at[l]], buf, gsem).wait()

        def fire_store(l, tb, ssem):
            pltpu.async_copy(tb, out_hbm.at[l, :, pl.ds(bcol, CHUNK)], ssem)

        def drain_store(tb, ssem):
            pltpu.make_async_copy(
                tb, out_hbm.at[0, :, pl.ds(bcol, CHUNK)], ssem).wait()

        def sel_transpose(l, buf, tb):
            def skew_row(g, _):
                offs = off_v[l, pl.ds(g * 16, 16)]
                for k16 in range(16):
                    off = offs[k16]
                    b = g * 16 + k16
                    base = b * K2_STRIDE
                    vs = [buf[b, pl.ds(off + 16 * t, 16)]
                          for t in range(D // 16)]
                    for t in range(D // 16):
                        pos = base + ((16 * t + b) & 63)
                        plsc.store_scatter(skew, [_iota16() + pos], vs[t])
                return 0

            lax.fori_loop(0, CHUNK // 16, skew_row, 0)

            def degroup(u, _):
                bv = _iota16() + 16 * u
                for d16 in range(0, D, 16):
                    hv = bv * K2_STRIDE + ((d16 + bv) & 63)
                    for d0 in range(0, 16, 8):
                        vecs = [plsc.load_gather(skew, [hv + dm])
                                for dm in range(d0, d0 + 8)]
                        for i, dm in enumerate(range(d0, d0 + 8)):
                            tb[d16 + dm, pl.ds(16 * u, 16)] = vecs[i]
                return 0

            lax.fori_loop(0, CHUNK // 16, degroup, 0)

        def halfstep(l, first, buf, gsem, tb, ssem, obuf, ogsem, otb, ossem):
            drain_gather(l, buf, gsem)
            if first:
                @pl.when(l >= 1)
                def _():
                    drain_store(otb, ossem)
            else:
                drain_store(otb, ossem)

            @pl.when(l + 1 < steps)
            def _():
                fire_gather(l + 1, obuf, ogsem)

            sel_transpose(l, buf, tb)
            fire_store(l, tb, ssem)

        fire_gather(0, buf0, gsem0)

        def body(t, _):
            halfstep(2 * t, True, buf0, gsem0, tb0, ssem0, buf1, gsem1, tb1, ssem1)
            halfstep(2 * t + 1, False, buf1, gsem1, tb1, ssem1, buf0, gsem0, tb0, ssem0)
            return 0

        lax.fori_loop(0, steps // 2, body, 0)
        drain_store(tb1, ssem1)

    return k


def kernel(x, table):
    B, L = x.shape
    V = table.shape[0]
    tpair = _make_k1(V)(jnp.swapaxes(table, 0, 1))
    xt = jnp.swapaxes(x, 0, 1).astype(jnp.int32)
    xpair = xt >> 1
    xoff = (xt & 1) * D
    out = _make_k2(B, L)(xpair, xoff, tpair)
    return jnp.transpose(out, (2, 0, 1))
